# Initial kernel scaffold; baseline (speedup 1.0000x reference)
import jax
import jax.numpy as jnp
from jax.experimental import pallas as pl


def kernel(x, edge_index, edge_attr, xe1, xe2, ee1, ee2, W1, b1, W2, b2, gamma, beta):
    raise NotImplementedError("write your pallas kernel here")


# speedup vs baseline: 2.9111x; 2.9111x over previous
import functools

import jax
import jax.numpy as jnp
from jax import lax
from jax.experimental import pallas as pl
from jax.experimental.pallas import tpu as pltpu
from jax.experimental.pallas import tpu_sc as plsc

NC = 2
NS = 16
NW = NC * NS
CH = 128

def _get_mesh():
    return plsc.VectorSubcoreMesh(core_axis_name="c", subcore_axis_name="s")


def _sc_aggregate(table, packed, zero_d, n_pad, d, k):
    @functools.partial(
        pl.kernel,
        out_type=jax.ShapeDtypeStruct((NC, n_pad, d), jnp.float32),
        mesh=_get_mesh(),
        scratch_types=[
            pltpu.VMEM((k, CH), jnp.int32),
            pltpu.VMEM((2, CH), jnp.int32),
            pltpu.VMEM((2, CH), jnp.int32),
            pltpu.VMEM((2, CH, d), jnp.float32),
            pltpu.VMEM_SHARED((n_pad, d), jnp.float32),
            pltpu.SemaphoreType.DMA,
            pltpu.SemaphoreType.DMA,
        ],
    )
    def agg(tab_hbm, packed_hbm, zero_hbm, out_hbm,
            pk_v, srcx_v, dstx_v, rows_v, acc_sh, sem0, sem1):
        c = lax.axis_index("c")
        s = lax.axis_index("s")
        stripe = n_pad // NS
        off = pl.multiple_of(s * stripe, 8)
        pltpu.sync_copy(zero_hbm.at[pl.ds(off, stripe)],
                        acc_sh.at[pl.ds(off, stripe)])
        pltpu.sync_copy(packed_hbm.at[c, s], pk_v)
        plsc.subcore_barrier()

        def unpack(ck, b):
            for g in range(CH // 16):
                v = pk_v[ck, pl.ds(g * 16, 16)]
                srcx_v[b, pl.ds(g * 16, 16)] = lax.shift_right_logical(v, 16)
                dstx_v[b, pl.ds(g * 16, 16)] = jnp.bitwise_and(v, 0xFFFF)

        sems = (sem0, sem1)
        for b in range(2):
            unpack(b, b)
            pltpu.async_copy(tab_hbm.at[srcx_v.at[b]], rows_v.at[b], sems[b])

        def body(kk, carry):
            for b in range(2):
                ck = kk * 2 + b
                pltpu.make_async_copy(tab_hbm.at[srcx_v.at[b]],
                                      rows_v.at[b], sems[b]).wait()
                pltpu.sync_copy(rows_v.at[b], acc_sh.at[dstx_v.at[b]], add=True)

                @pl.when(ck + 2 < k)
                def _():
                    unpack(ck + 2, b)
                    pltpu.async_copy(tab_hbm.at[srcx_v.at[b]],
                                     rows_v.at[b], sems[b])
            return carry

        lax.fori_loop(0, k // 2, body, 0)
        plsc.subcore_barrier()
        pltpu.sync_copy(acc_sh.at[pl.ds(off, stripe)],
                        out_hbm.at[c, pl.ds(off, stripe)])

    return agg(table, packed, zero_d)


def _h0(x, xe1p, xe2p, n, d):
    t1 = xe1p.shape[0]
    t2 = xe2p.shape[0]

    def body(x_ref, e1_ref, e2_ref, o_ref):
        x0 = x_ref[:, 0:1]
        x1 = x_ref[:, 1:2]
        oh0 = (x0 == lax.broadcasted_iota(jnp.int32, (1, t1), 1)).astype(jnp.float32)
        oh1 = (x1 == lax.broadcasted_iota(jnp.int32, (1, t2), 1)).astype(jnp.float32)
        o_ref[...] = (jnp.dot(oh0, e1_ref[...], preferred_element_type=jnp.float32, precision=lax.Precision.HIGHEST)
                      + jnp.dot(oh1, e2_ref[...], preferred_element_type=jnp.float32, precision=lax.Precision.HIGHEST))

    return pl.pallas_call(
        body, out_shape=jax.ShapeDtypeStruct((n, d), jnp.float32))(x, xe1p, xe2p)


def _layer(part, cnt, eel, w1, b1, w2, b2, g, be, relu_out, n, d):
    def body(p_ref, c_ref, ee_ref, w1_ref, b1_ref, w2_ref, b2_ref,
             g_ref, be_ref, o_ref):
        aggr = p_ref[0, :n, :] + p_ref[1, :n, :]
        cm = c_ref[0, :n, :] + c_ref[1, :n, :]
        aggr = aggr + jnp.dot(cm, ee_ref[...], preferred_element_type=jnp.float32,
                              precision=lax.Precision.HIGHEST)
        h1 = jnp.dot(aggr, w1_ref[...], preferred_element_type=jnp.float32) + b1_ref[...]
        h1 = jnp.maximum(h1, 0.0)
        h2 = jnp.dot(h1, w2_ref[...], preferred_element_type=jnp.float32) + b2_ref[...]
        mu = jnp.mean(h2, axis=0, keepdims=True)
        var = jnp.mean((h2 - mu) ** 2, axis=0, keepdims=True)
        out = g_ref[...] * (h2 - mu) / jnp.sqrt(var + 1e-5) + be_ref[...]
        if relu_out:
            out = jnp.maximum(out, 0.0)
        o_ref[...] = out

    return pl.pallas_call(
        body, out_shape=jax.ShapeDtypeStruct((n, d), jnp.float32))(
            part, cnt, eel, w1, b1, w2, b2, g, be)


def kernel(x, edge_index, edge_attr, xe1, xe2, ee1, ee2, W1, b1, W2, b2, gamma, beta):
    n = x.shape[0]
    e = edge_index.shape[1]
    d = xe1.shape[1]
    nl = W1.shape[0]
    nb = ee1.shape[1]
    nr = ee2.shape[1]

    n_pad = -(-(n + 1) // (NS * 8)) * (NS * 8)
    k = -(-e // (NW * CH))
    k += k % 2
    epad = NW * CH * k
    pad_e = epad - e

    src = edge_index[0].astype(jnp.int32)
    dst = edge_index[1].astype(jnp.int32)
    dstp = jnp.concatenate([dst, jnp.full((pad_e,), n, jnp.int32)])
    srcp = jnp.concatenate([src, jnp.zeros((pad_e,), jnp.int32)])
    packed_h = ((srcp << 16) | dstp).reshape(NC, NS, k, CH)

    tid = edge_attr[:, 0].astype(jnp.int32) * nr + edge_attr[:, 1].astype(jnp.int32)
    tidp = jnp.concatenate([tid, jnp.full((pad_e,), nb * nr, jnp.int32)])
    packed_c = ((tidp << 16) | dstp).reshape(NC, NS, k, CH)

    tt = jnp.arange(nb * nr, dtype=jnp.int32)
    t18 = (jax.nn.one_hot(tt // nr, d, dtype=jnp.float32)
           + jax.nn.one_hot(nb + tt % nr, d, dtype=jnp.float32))
    t18 = jnp.concatenate([t18, jnp.zeros((32 - nb * nr, d), jnp.float32)])

    zero_d = jnp.zeros((n_pad, d), jnp.float32)

    xe1p = jnp.concatenate([xe1, jnp.zeros((128 - xe1.shape[0], d), jnp.float32)])
    xe2p = jnp.concatenate([xe2, jnp.zeros((8 - xe2.shape[0], d), jnp.float32)])

    h = _h0(x.astype(jnp.int32), xe1p, xe2p, n, d)
    cnt = _sc_aggregate(t18, packed_c, zero_d, n_pad, d, k)

    for l in range(nl):
        eel = jnp.concatenate(
            [ee1[l], ee2[l], jnp.zeros((d - nb - nr, d), jnp.float32)], axis=0)
        part = _sc_aggregate(h, packed_h, zero_d, n_pad, d, k)
        h = _layer(part, cnt, eel, W1[l], b1[l][None, :], W2[l], b2[l][None, :],
                   gamma[l][None, :], beta[l][None, :], l < nl - 1, n, d)
    return h
